# baseline (device time: 120621 ns/iter reference)
import os

import jax
import jax.numpy as jnp
from jax import lax
from jax.experimental import pallas as pl
from jax.experimental.pallas import tpu as pltpu

_SKIP_COMM = os.environ.get("KERNEL_SKIP_COMM") == "1"
_SKIP_GEMM = os.environ.get("KERNEL_SKIP_GEMM") == "1"
_SKIP_X = os.environ.get("KERNEL_SKIP_X") == "1"
_ONLY_LOAD = os.environ.get("KERNEL_ONLY_LOAD") == "1"
_SKIP_VPU = os.environ.get("KERNEL_SKIP_VPU") == "1"

M, K, N = 2048, 8192, 2048
C = int(os.environ.get("KERNEL_C", "16"))
CN = N // C
S = int(os.environ.get("KERNEL_S", "4"))
HALF = M // 2


def kernel(dy, W):
    def body(dy_ref, w_ref, out_ref, dy_v, dy_land, w_buf, w_land, acc,
             y_recv, x_send_bf, x_recv,
             dy_sems, w_sems, y_send_sems, y_recv_sems,
             x_send_sems, x_recv_sems, y_credit_sem, x_credit_sem):
        my_x = lax.axis_index("x")
        my_y = lax.axis_index("y")
        y_peer = (my_x, 1 - my_y)
        x_peer = (1 - my_x, my_y)
        rows = pl.ds(my_x * HALF, HALF)
        other_rows = pl.ds((1 - my_x) * HALF, HALF)

        def w_copies(c):
            slot = lax.rem(c, 2)
            h = CN // 2
            return [
                pltpu.make_async_copy(
                    w_ref.at[pl.ds(c * CN + i * h, h), :],
                    w_land.at[slot, pl.ds(i * h, h), :],
                    w_sems.at[slot, i])
                for i in range(2)
            ]

        def y_rdma(c):
            slot = lax.rem(c, S)
            return pltpu.make_async_remote_copy(
                src_ref=acc.at[slot], dst_ref=y_recv.at[slot],
                send_sem=y_send_sems.at[c], recv_sem=y_recv_sems.at[c],
                device_id=y_peer, device_id_type=pl.DeviceIdType.MESH)

        def x_rdma(c):
            slot = lax.rem(c, S)
            return pltpu.make_async_remote_copy(
                src_ref=x_send_bf.at[slot], dst_ref=x_recv.at[slot],
                send_sem=x_send_sems.at[c], recv_sem=x_recv_sems.at[c],
                device_id=x_peer, device_id_type=pl.DeviceIdType.MESH)

        DY_STREAMS = 8
        RB = HALF // DY_STREAMS

        def dy_copy(i):
            return pltpu.make_async_copy(
                dy_ref.at[pl.ds(my_x * HALF + i * RB, RB), :],
                dy_land.at[i % 2], dy_sems.at[i % 2])

        if not _SKIP_GEMM:
            dy_copy(0).start()
            dy_copy(1).start()
            for cp in w_copies(0) + w_copies(1):
                cp.start()

        if not _SKIP_COMM:
            barrier_sem = pltpu.get_barrier_semaphore()
            for peer in (y_peer, x_peer):
                pl.semaphore_signal(
                    barrier_sem, inc=1, device_id=peer,
                    device_id_type=pl.DeviceIdType.MESH)
            pl.semaphore_wait(barrier_sem, 2)

        if not _SKIP_GEMM:
            for i in range(DY_STREAMS):
                dy_copy(i).wait()
                if i + 2 < DY_STREAMS:
                    dy_copy(i + 2).start()
                dy_v[pl.ds(i * RB, RB), :] = dy_land[i % 2].astype(
                    jnp.bfloat16)

        if _ONLY_LOAD:
            for cp in w_copies(0) + w_copies(1):
                cp.wait()
            out_ref[rows, :] = (
                dy_v[:, :N] + w_land[0, 0, :N].astype(jnp.bfloat16)[None, :]
            ).astype(jnp.float32)
            out_ref[other_rows, :] = dy_v[:, :N].astype(jnp.float32)
            return

        def consume_y(d):
            slot = lax.rem(d, S)
            cols = pl.ds(d * CN, CN)
            yr = y_rdma(d)
            yr.wait_send()
            yr.wait_recv()
            if not _SKIP_VPU:
                x_send_bf[slot] = acc[slot] + y_recv[slot]
                out_ref[rows, cols] = x_send_bf[slot].astype(jnp.float32)

            @pl.when(d + S < C)
            def _():
                pl.semaphore_signal(
                    y_credit_sem, inc=1, device_id=y_peer,
                    device_id_type=pl.DeviceIdType.MESH)

            if not _SKIP_X:
                @pl.when(d >= S)
                def _():
                    pl.semaphore_wait(x_credit_sem, 1)

                x_rdma(d).start()

        def consume_x(d):
            slot = lax.rem(d, S)
            cols = pl.ds(d * CN, CN)
            x_rdma(d).wait_recv()
            if not _SKIP_VPU:
                out_ref[other_rows, cols] = x_recv[slot].astype(jnp.float32)

            @pl.when(d + S < C)
            def _():
                pl.semaphore_signal(
                    x_credit_sem, inc=1, device_id=x_peer,
                    device_id_type=pl.DeviceIdType.MESH)

        def step(c, carry):
            slot = lax.rem(c, S)

            if not _SKIP_COMM and not _SKIP_X:
                @pl.when(c >= S)
                def _():
                    x_rdma(c - S).wait_send()

            if _SKIP_GEMM:
                acc[slot] = jnp.zeros((HALF, CN), jnp.bfloat16)
            else:
                wslot = lax.rem(c, 2)
                for cp in w_copies(c):
                    cp.wait()
                w_buf[wslot] = w_land[wslot].astype(jnp.bfloat16)

                @pl.when(c + 2 < C)
                def _():
                    for cp in w_copies(c + 2):
                        cp.start()

                p = lax.dot_general(
                    dy_v[...], w_buf[wslot],
                    dimension_numbers=(((1,), (1,)), ((), ())),
                    preferred_element_type=jnp.float32,
                ).astype(jnp.bfloat16)
                acc[slot] = p

            if _SKIP_COMM:
                out_ref[rows, pl.ds(c * CN, CN)] = acc[slot].astype(
                    jnp.float32)
            else:
                @pl.when(c >= S)
                def _():
                    pl.semaphore_wait(y_credit_sem, 1)

                y_rdma(c).start()

                @pl.when(c >= 2)
                def _():
                    consume_y(c - 2)

                if not _SKIP_X:
                    @pl.when(c >= 4)
                    def _():
                        consume_x(c - 4)

            return carry

        lax.fori_loop(0, C, step, 0)

        if not _SKIP_COMM:
            consume_y(C - 2)
            consume_y(C - 1)
            if not _SKIP_X:
                for d in range(C - 4, C):
                    consume_x(d)

                def wtail(c, carry):
                    x_rdma(c).wait_send()
                    return carry

                lax.fori_loop(C - S, C, wtail, 0)

    return pl.pallas_call(
        body,
        out_shape=jax.ShapeDtypeStruct((M, N), jnp.float32),
        in_specs=[
            pl.BlockSpec(memory_space=pltpu.MemorySpace.HBM),
            pl.BlockSpec(memory_space=pltpu.MemorySpace.HBM),
        ],
        out_specs=pl.BlockSpec(memory_space=pltpu.VMEM),
        scratch_shapes=[
            pltpu.VMEM((8, 8) if _SKIP_GEMM else (HALF, K), jnp.bfloat16),
            pltpu.VMEM(
                (2, 8, 8) if _SKIP_GEMM else (2, HALF // 8, K),
                jnp.float32),
            pltpu.VMEM((2, 8, 8) if _SKIP_GEMM else (2, CN, K),
                       jnp.bfloat16),
            pltpu.VMEM((2, 8, 8) if _SKIP_GEMM else (2, CN, K),
                       jnp.float32),
            pltpu.VMEM((S, HALF, CN), jnp.bfloat16),
            pltpu.VMEM((S, HALF, CN), jnp.bfloat16),
            pltpu.VMEM((S, HALF, CN), jnp.bfloat16),
            pltpu.VMEM((S, HALF, CN), jnp.bfloat16),
            pltpu.SemaphoreType.DMA((8,)),
            pltpu.SemaphoreType.DMA((2, 2)),
            pltpu.SemaphoreType.DMA((C,)),
            pltpu.SemaphoreType.DMA((C,)),
            pltpu.SemaphoreType.DMA((C,)),
            pltpu.SemaphoreType.DMA((C,)),
            pltpu.SemaphoreType.REGULAR,
            pltpu.SemaphoreType.REGULAR,
        ],
        compiler_params=pltpu.CompilerParams(
            collective_id=None if _SKIP_COMM else 0,
            vmem_limit_bytes=64 * 1024 * 1024,
        ),
    )(dy, W)


# device time: 120013 ns/iter; 1.0051x vs baseline; 1.0051x over previous
import os

import jax
import jax.numpy as jnp
from jax import lax
from jax.experimental import pallas as pl
from jax.experimental.pallas import tpu as pltpu

_SKIP_COMM = os.environ.get("KERNEL_SKIP_COMM") == "1"
_SKIP_GEMM = os.environ.get("KERNEL_SKIP_GEMM") == "1"
_SKIP_X = os.environ.get("KERNEL_SKIP_X") == "1"
_ONLY_LOAD = os.environ.get("KERNEL_ONLY_LOAD") == "1"
_SKIP_VPU = os.environ.get("KERNEL_SKIP_VPU") == "1"

M, K, N = 2048, 8192, 2048
C = int(os.environ.get("KERNEL_C", "16"))
CN = N // C
S = int(os.environ.get("KERNEL_S", "4"))
HALF = M // 2


def kernel(dy, W):
    def body(dy_ref, w_ref, out_ref, dy_v, w_buf, acc,
             y_recv, x_send_bf, x_recv,
             dy_sems, w_sems, y_send_sems, y_recv_sems,
             x_send_sems, x_recv_sems, y_credit_sem, x_credit_sem):
        my_x = lax.axis_index("x")
        my_y = lax.axis_index("y")
        y_peer = (my_x, 1 - my_y)
        x_peer = (1 - my_x, my_y)
        rows = pl.ds(my_x * HALF, HALF)
        other_rows = pl.ds((1 - my_x) * HALF, HALF)

        def w_copies(c):
            slot = lax.rem(c, 2)
            h = CN // 2
            return [
                pltpu.make_async_copy(
                    w_ref.at[pl.ds(c * CN + i * h, h), :],
                    w_buf.at[slot, pl.ds(i * h, h), :],
                    w_sems.at[slot, i])
                for i in range(2)
            ]

        def y_rdma(c):
            slot = lax.rem(c, S)
            return pltpu.make_async_remote_copy(
                src_ref=acc.at[slot], dst_ref=y_recv.at[slot],
                send_sem=y_send_sems.at[c], recv_sem=y_recv_sems.at[c],
                device_id=y_peer, device_id_type=pl.DeviceIdType.MESH)

        def x_rdma(c):
            slot = lax.rem(c, S)
            return pltpu.make_async_remote_copy(
                src_ref=x_send_bf.at[slot], dst_ref=x_recv.at[slot],
                send_sem=x_send_sems.at[c], recv_sem=x_recv_sems.at[c],
                device_id=x_peer, device_id_type=pl.DeviceIdType.MESH)

        DY_STREAMS = 8
        dy_copies = []
        if not _SKIP_GEMM:
            rb = HALF // DY_STREAMS
            for i in range(DY_STREAMS):
                dy_copies.append(pltpu.make_async_copy(
                    dy_ref.at[pl.ds(my_x * HALF + i * rb, rb), :],
                    dy_v.at[pl.ds(i * rb, rb), :],
                    dy_sems.at[i]))
                dy_copies[-1].start()
            for cp in w_copies(0) + w_copies(1):
                cp.start()

        if not _SKIP_COMM:
            barrier_sem = pltpu.get_barrier_semaphore()
            for peer in (y_peer, x_peer):
                pl.semaphore_signal(
                    barrier_sem, inc=1, device_id=peer,
                    device_id_type=pl.DeviceIdType.MESH)
            pl.semaphore_wait(barrier_sem, 2)

        if not _SKIP_GEMM:
            for cp in dy_copies:
                cp.wait()

        if _ONLY_LOAD:
            for cp in w_copies(0) + w_copies(1):
                cp.wait()
            out_ref[rows, :] = dy_v[:, :N] + w_buf[0, 0, :N][None, :]
            out_ref[other_rows, :] = dy_v[:, :N]
            return

        def consume_y(d):
            slot = lax.rem(d, S)
            cols = pl.ds(d * CN, CN)
            yr = y_rdma(d)
            yr.wait_send()
            yr.wait_recv()
            if not _SKIP_VPU:
                x_send_bf[slot] = acc[slot] + y_recv[slot]
                out_ref[rows, cols] = x_send_bf[slot].astype(jnp.float32)

            @pl.when(d + S < C)
            def _():
                pl.semaphore_signal(
                    y_credit_sem, inc=1, device_id=y_peer,
                    device_id_type=pl.DeviceIdType.MESH)

            if not _SKIP_X:
                @pl.when(d >= S)
                def _():
                    pl.semaphore_wait(x_credit_sem, 1)

                x_rdma(d).start()

        def consume_x(d):
            slot = lax.rem(d, S)
            cols = pl.ds(d * CN, CN)
            x_rdma(d).wait_recv()
            if not _SKIP_VPU:
                out_ref[other_rows, cols] = x_recv[slot].astype(jnp.float32)

            @pl.when(d + S < C)
            def _():
                pl.semaphore_signal(
                    x_credit_sem, inc=1, device_id=x_peer,
                    device_id_type=pl.DeviceIdType.MESH)

        def step(c, carry):
            slot = lax.rem(c, S)

            if not _SKIP_COMM and not _SKIP_X:
                @pl.when(c >= S)
                def _():
                    x_rdma(c - S).wait_send()

            if _SKIP_GEMM:
                acc[slot] = jnp.zeros((HALF, CN), jnp.bfloat16)
            else:
                for cp in w_copies(c):
                    cp.wait()
                p = lax.dot_general(
                    dy_v[...], w_buf[lax.rem(c, 2)],
                    dimension_numbers=(((1,), (1,)), ((), ())),
                    preferred_element_type=jnp.float32,
                ).astype(jnp.bfloat16)

                @pl.when(c + 2 < C)
                def _():
                    for cp in w_copies(c + 2):
                        cp.start()

                acc[slot] = p

            if _SKIP_COMM:
                out_ref[rows, pl.ds(c * CN, CN)] = acc[slot].astype(
                    jnp.float32)
            else:
                @pl.when(c >= S)
                def _():
                    pl.semaphore_wait(y_credit_sem, 1)

                y_rdma(c).start()

                @pl.when(c >= 2)
                def _():
                    consume_y(c - 2)

                if not _SKIP_X:
                    @pl.when(c >= 4)
                    def _():
                        consume_x(c - 4)

            return carry

        lax.fori_loop(0, C, step, 0)

        if not _SKIP_COMM:
            consume_y(C - 2)
            consume_y(C - 1)
            if not _SKIP_X:
                for d in range(C - 4, C):
                    consume_x(d)

                def wtail(c, carry):
                    x_rdma(c).wait_send()
                    return carry

                lax.fori_loop(C - S, C, wtail, 0)

    return pl.pallas_call(
        body,
        out_shape=jax.ShapeDtypeStruct((M, N), jnp.float32),
        in_specs=[
            pl.BlockSpec(memory_space=pltpu.MemorySpace.HBM),
            pl.BlockSpec(memory_space=pltpu.MemorySpace.HBM),
        ],
        out_specs=pl.BlockSpec(memory_space=pltpu.VMEM),
        scratch_shapes=[
            pltpu.VMEM((8, 8) if _SKIP_GEMM else (HALF, K), jnp.float32),
            pltpu.VMEM((2, 8, 8) if _SKIP_GEMM else (2, CN, K), jnp.float32),
            pltpu.VMEM((S, HALF, CN), jnp.bfloat16),
            pltpu.VMEM((S, HALF, CN), jnp.bfloat16),
            pltpu.VMEM((S, HALF, CN), jnp.bfloat16),
            pltpu.VMEM((S, HALF, CN), jnp.bfloat16),
            pltpu.SemaphoreType.DMA((8,)),
            pltpu.SemaphoreType.DMA((2, 2)),
            pltpu.SemaphoreType.DMA((C,)),
            pltpu.SemaphoreType.DMA((C,)),
            pltpu.SemaphoreType.DMA((C,)),
            pltpu.SemaphoreType.DMA((C,)),
            pltpu.SemaphoreType.REGULAR,
            pltpu.SemaphoreType.REGULAR,
        ],
        compiler_params=pltpu.CompilerParams(
            collective_id=None if _SKIP_COMM else 0,
            vmem_limit_bytes=64 * 1024 * 1024,
        ),
    )(dy, W)
